# baseline (speedup 1.0000x reference)
import functools

import jax
import jax.numpy as jnp
from jax import lax
from jax.experimental import pallas as pl
from jax.experimental.pallas import tpu as pltpu
from jax.experimental.pallas import tpu_sc as plsc

_NC = 2
_NS = 16
_NW = _NC * _NS
_CHUNK = 16


def _sc_body(seq_hbm, lens_hbm, tab_hbm, out_hbm, buf, rows, idx_v, len_v, sem):
    wid = lax.axis_index("s") * _NC + lax.axis_index("c")
    n_rows = seq_hbm.shape[0]
    rows_per_w = n_rows // _NW
    L = 2048
    row0 = wid * rows_per_w
    b = row0 // L

    idx_v[...] = jnp.full((16,), b, dtype=jnp.int32)
    pltpu.async_copy(lens_hbm.at[idx_v], len_v, sem).wait()
    lenvec = len_v[...]
    l0 = row0 % L
    D = buf.shape[1]

    def step(c, carry):
        base = row0 + c * _CHUNK
        lpos = lax.iota(jnp.int32, 16) + (l0 + c * _CHUNK)
        ids = jnp.where(lpos < lenvec, lpos + 1, 0)
        idx_v[...] = ids
        pltpu.sync_copy(seq_hbm.at[pl.ds(base, _CHUNK)], buf)
        pltpu.async_copy(tab_hbm.at[idx_v], rows, sem).wait()

        def add_row(r, carry2):
            for j in range(D // 16):
                o = j * 16
                buf[r, pl.ds(o, 16)] = buf[r, pl.ds(o, 16)] + rows[r, pl.ds(o, 16)]
            return carry2

        lax.fori_loop(0, _CHUNK, add_row, 0)
        pltpu.sync_copy(buf, out_hbm.at[pl.ds(base, _CHUNK)])
        return carry

    lax.fori_loop(0, rows_per_w // _CHUNK, step, 0)


def kernel(sequence, sequence_lenghts, pos_table):
    B, L, D = sequence.shape
    seq_flat = sequence.reshape(B * L, D)
    lens = sequence_lenghts.astype(jnp.int32)

    k = functools.partial(
        pl.kernel,
        out_type=jax.ShapeDtypeStruct((B * L, D), jnp.float32),
        mesh=plsc.VectorSubcoreMesh(core_axis_name="c", subcore_axis_name="s"),
        scratch_types=[
            pltpu.VMEM((_CHUNK, D), jnp.float32),
            pltpu.VMEM((_CHUNK, D), jnp.float32),
            pltpu.VMEM((16,), jnp.int32),
            pltpu.VMEM((16,), jnp.int32),
            pltpu.SemaphoreType.DMA,
        ],
    )(_sc_body)
    out_flat = k(seq_flat, lens, pos_table)
    return out_flat.reshape(B, L, D)
